# baseline (speedup 1.0000x reference)
import jax
import jax.numpy as jnp
from jax.experimental import pallas as pl
from jax.experimental.pallas import tpu as pltpu

_NUM_CLASS = 1000
_TILE_M = 1024


def _cdiv(a: int, b: int) -> int:
    return (a + b - 1) // b


def _linear_kernel(x_ref, w_ref, b_ref, o_ref):
    xb = x_ref[...].astype(jnp.bfloat16)
    wb = w_ref[...].astype(jnp.bfloat16)
    acc = jnp.dot(xb, wb, preferred_element_type=jnp.float32)
    o_ref[...] = (acc + b_ref[...]).astype(o_ref.dtype)


def kernel(x, w_t, b):
    B, D = x.shape
    Dw, Cp = w_t.shape
    assert D == Dw and _NUM_CLASS <= Cp

    tile_m = min(_TILE_M, B)
    grid = (_cdiv(B, tile_m),)
    out = pl.pallas_call(
        _linear_kernel,
        out_shape=jax.ShapeDtypeStruct((B, Cp), x.dtype),
        grid=grid,
        in_specs=[
            pl.BlockSpec((tile_m, D), lambda i: (i, 0)),
            pl.BlockSpec((D, Cp), lambda i: (0, 0)),
            pl.BlockSpec((1, Cp), lambda i: (0, 0)),
        ],
        out_specs=pl.BlockSpec((tile_m, Cp), lambda i: (i, 0)),
        compiler_params=pltpu.CompilerParams(
            dimension_semantics=("parallel",)),
    )(x, w_t, b)
    return out[:, :_NUM_CLASS]


# speedup vs baseline: 1.1291x; 1.1291x over previous
import jax
import jax.numpy as jnp
from jax.experimental import pallas as pl
from jax.experimental.pallas import tpu as pltpu

_NUM_CLASS = 1000
_TILE_M = 1024
_K_SLABS = 4


def _cdiv(a: int, b: int) -> int:
    return (a + b - 1) // b


def _linear_kernel(x0_ref, x1_ref, x2_ref, x3_ref, w_ref, b_ref, o_ref):
    wb = w_ref[...].astype(jnp.bfloat16)
    ks = w_ref.shape[0] // _K_SLABS
    acc = jnp.dot(x0_ref[...].astype(jnp.bfloat16), wb[:ks],
                  preferred_element_type=jnp.float32)
    acc += jnp.dot(x1_ref[...].astype(jnp.bfloat16), wb[ks:2 * ks],
                   preferred_element_type=jnp.float32)
    acc += jnp.dot(x2_ref[...].astype(jnp.bfloat16), wb[2 * ks:3 * ks],
                   preferred_element_type=jnp.float32)
    acc += jnp.dot(x3_ref[...].astype(jnp.bfloat16), wb[3 * ks:],
                   preferred_element_type=jnp.float32)
    out = acc + b_ref[...]
    o_ref[...] = out[:, :_NUM_CLASS].astype(o_ref.dtype)


def kernel(x, w_t, b):
    B, D = x.shape
    Dw, Cp = w_t.shape
    assert D == Dw and _NUM_CLASS <= Cp

    tile_m = min(_TILE_M, B)
    ks = D // _K_SLABS
    grid = (_cdiv(B, tile_m),)

    def slab_spec(j):
        return pl.BlockSpec((tile_m, ks), lambda i, j=j: (i, j))

    return pl.pallas_call(
        _linear_kernel,
        out_shape=jax.ShapeDtypeStruct((B, _NUM_CLASS), x.dtype),
        grid=grid,
        in_specs=[
            slab_spec(0),
            slab_spec(1),
            slab_spec(2),
            slab_spec(3),
            pl.BlockSpec((D, Cp), lambda i: (0, 0)),
            pl.BlockSpec((1, Cp), lambda i: (0, 0)),
        ],
        out_specs=pl.BlockSpec((tile_m, _NUM_CLASS), lambda i: (i, 0)),
        compiler_params=pltpu.CompilerParams(
            dimension_semantics=("arbitrary",)),
    )(x, x, x, x, w_t, b)
